# baseline (speedup 1.0000x reference)
import functools

import jax
import jax.numpy as jnp
from jax import lax
from jax.experimental import pallas as pl
from jax.experimental.pallas import tpu as pltpu
from jax.experimental.pallas import tpu_sc as plsc

S, D, E, H = 2048, 768, 8, 1024
EP = 128
TILE = 1024
P = 2 * S + E * TILE
NT = P // TILE
NC, NS, L = 2, 16, 16
NW = NC * NS
TPW = S // NW
NEG = -1e30

_INV_SQRT2 = 0.7071067811865476


def _gelu_exact(x):
    z = x * _INV_SQRT2
    az = jnp.abs(z)
    t = 1.0 / (1.0 + 0.3275911 * az)
    poly = ((((1.061405429 * t - 1.453152027) * t + 1.421413741) * t
             - 0.284496736) * t + 0.254829592) * t
    erf_abs = 1.0 - poly * jnp.exp(-az * az)
    erf = jnp.where(z < 0, -erf_abs, erf_abs)
    return 0.5 * x * (1.0 + erf)


def _route_body(x_ref, wg_ref, pos0_ref, pos1_ref, pr_ref, eot_ref):
    s = jnp.dot(x_ref[...], wg_ref[...], preferred_element_type=jnp.float32)
    lane = lax.broadcasted_iota(jnp.int32, (S, EP), 1)
    s = jnp.where(lane < E, s, NEG)
    m1 = jnp.max(s, axis=1, keepdims=True)
    i1 = jnp.min(jnp.where(s >= m1, lane, EP), axis=1, keepdims=True)
    oh1 = lane == i1
    s2 = jnp.where(oh1, NEG, s)
    m2 = jnp.max(s2, axis=1, keepdims=True)
    i2 = jnp.min(jnp.where(s2 >= m2, lane, EP), axis=1, keepdims=True)
    oh2 = lane == i2
    p1 = 1.0 / (1.0 + jnp.exp(m2 - m1))
    p2 = 1.0 - p1

    cnt = oh1.astype(jnp.float32) + oh2.astype(jnp.float32)
    G = 16
    GS = S // G
    cnt3 = cnt.reshape(G, GS, EP)
    ri = lax.broadcasted_iota(jnp.int32, (GS, GS), 0)
    ci = lax.broadcasted_iota(jnp.int32, (GS, GS), 1)
    l_incl = (ri >= ci).astype(jnp.float32)
    incl = [jnp.dot(l_incl, cnt3[g], preferred_element_type=jnp.float32)
            for g in range(G)]
    tot = jnp.concatenate([incl[g][GS - 1:GS, :] for g in range(G)], axis=0)
    ri2 = lax.broadcasted_iota(jnp.int32, (G, G), 0)
    ci2 = lax.broadcasted_iota(jnp.int32, (G, G), 1)
    m_strict = (ci2 < ri2).astype(jnp.float32)
    go = jnp.dot(m_strict, tot, preferred_element_type=jnp.float32)
    excl = jnp.concatenate(
        [go[g:g + 1, :] + incl[g] - cnt3[g] for g in range(G)], axis=0)

    c = go[G - 1:G, :] + tot[G - 1:G, :]
    pc = jnp.ceil(c * (1.0 / TILE)) * TILE
    ri3 = lax.broadcasted_iota(jnp.int32, (EP, EP), 0)
    ci3 = lax.broadcasted_iota(jnp.int32, (EP, EP), 1)
    m_lt = (ri3 < ci3).astype(jnp.float32)
    po = jnp.dot(pc, m_lt, preferred_element_type=jnp.float32)

    pos0 = jnp.sum(jnp.where(oh1, po + excl, 0.0), axis=1, keepdims=True)
    pos1 = jnp.sum(jnp.where(oh2, po + excl, 0.0), axis=1, keepdims=True)
    pos0_ref[...] = pos0.astype(jnp.int32)
    pos1_ref[...] = pos1.astype(jnp.int32)
    pr_ref[...] = jnp.where(lane < 64, p1, p2)

    jt = (lax.broadcasted_iota(jnp.int32, (EP, EP), 0) * TILE).astype(jnp.float32)
    ind = (po <= jt) & (ci3 < E)
    eot_col = jnp.sum(ind.astype(jnp.float32), axis=1, keepdims=True) - 1.0
    pt = jnp.sum(jnp.where(ci3[0:1] < E, pc, 0.0), axis=1, keepdims=True)
    eot_col = jnp.where(jt[:, 0:1] < pt, eot_col, eot_col + E)
    eot_ref[...] = jnp.broadcast_to(eot_col, (EP, 8)).astype(jnp.int32)


def _route(x2, wg_p):
    return pl.pallas_call(
        _route_body,
        out_shape=(
            jax.ShapeDtypeStruct((S, 1), jnp.int32),
            jax.ShapeDtypeStruct((S, 1), jnp.int32),
            jax.ShapeDtypeStruct((S, EP), jnp.float32),
            jax.ShapeDtypeStruct((EP, 8), jnp.int32),
        ),
    )(x2, wg_p)


@functools.cache
def _sc_mesh():
    return plsc.VectorSubcoreMesh(core_axis_name="c", subcore_axis_name="s")


@functools.cache
def _dispatch_kernel():
    @functools.partial(
        pl.kernel,
        out_type=jax.ShapeDtypeStruct((P, D), jnp.float32),
        mesh=_sc_mesh(),
        scratch_types=[
            pltpu.VMEM((TPW,), jnp.int32),
            pltpu.VMEM((TPW,), jnp.int32),
            pltpu.VMEM((TPW, D), jnp.float32),
            pltpu.SemaphoreType.DMA,
        ],
    )
    def _dispatch(x_hbm, i0_hbm, i1_hbm, xs_hbm, i0_v, i1_v, rows_v, sem):
        w = lax.axis_index("s") * NC + lax.axis_index("c")
        pltpu.sync_copy(x_hbm.at[pl.ds(w * TPW, TPW)], rows_v)
        pltpu.sync_copy(i0_hbm.at[w], i0_v)
        pltpu.sync_copy(i1_hbm.at[w], i1_v)
        c0 = pltpu.async_copy(rows_v, xs_hbm.at[i0_v], sem)
        c1 = pltpu.async_copy(rows_v, xs_hbm.at[i1_v], sem)
        c0.wait()
        c1.wait()

    return _dispatch


def _ffn_body(eot_ref, xs_ref, w1_ref, b1_ref, w2_ref, b2_ref, ys_ref):
    e = eot_ref[pl.program_id(0), 0]

    @pl.when(e < E)
    def _():
        h = jnp.dot(xs_ref[...], w1_ref[0], preferred_element_type=jnp.float32)
        h = _gelu_exact(h + b1_ref[0])
        ys_ref[...] = (
            jnp.dot(h, w2_ref[0], preferred_element_type=jnp.float32)
            + b2_ref[0])


def _safe(e):
    return e % E


def _ffn(eot, xs, W1, b1r, W2, b2r):
    grid_spec = pltpu.PrefetchScalarGridSpec(
        num_scalar_prefetch=1,
        grid=(NT,),
        in_specs=[
            pl.BlockSpec((TILE, D), lambda i, eot: (i, 0)),
            pl.BlockSpec((1, D, H), lambda i, eot: (_safe(eot[i, 0]), 0, 0)),
            pl.BlockSpec((1, 1, H), lambda i, eot: (_safe(eot[i, 0]), 0, 0)),
            pl.BlockSpec((1, H, D), lambda i, eot: (_safe(eot[i, 0]), 0, 0)),
            pl.BlockSpec((1, 1, D), lambda i, eot: (_safe(eot[i, 0]), 0, 0)),
        ],
        out_specs=pl.BlockSpec((TILE, D), lambda i, eot: (i, 0)),
    )
    return pl.pallas_call(
        _ffn_body,
        grid_spec=grid_spec,
        out_shape=jax.ShapeDtypeStruct((P, D), jnp.float32),
    )(eot, xs, W1, b1r, W2, b2r)


@functools.cache
def _combine_kernel():
    @functools.partial(
        pl.kernel,
        out_type=jax.ShapeDtypeStruct((S, D), jnp.float32),
        mesh=_sc_mesh(),
        scratch_types=[
            pltpu.VMEM((TPW,), jnp.int32),
            pltpu.VMEM((TPW,), jnp.int32),
            pltpu.VMEM((TPW, D), jnp.float32),
            pltpu.VMEM((TPW, D), jnp.float32),
            pltpu.VMEM((TPW, EP), jnp.float32),
            pltpu.SemaphoreType.DMA,
        ],
    )
    def _combine(ys_hbm, i0_hbm, i1_hbm, pr_hbm, o_hbm,
                 i0_v, i1_v, g0_v, g1_v, pr_v, sem):
        w = lax.axis_index("s") * NC + lax.axis_index("c")
        pltpu.sync_copy(i0_hbm.at[w], i0_v)
        pltpu.sync_copy(i1_hbm.at[w], i1_v)
        pltpu.sync_copy(pr_hbm.at[pl.ds(w * TPW, TPW)], pr_v)
        c0 = pltpu.async_copy(ys_hbm.at[i0_v], g0_v, sem)
        c1 = pltpu.async_copy(ys_hbm.at[i1_v], g1_v, sem)
        c0.wait()
        c1.wait()

        def body(t, carry):
            vp0 = pr_v[t, pl.ds(0, L)]
            vp1 = pr_v[t, pl.ds(64, L)]
            for ch in range(D // L):
                sl = pl.ds(ch * L, L)
                g0_v[t, sl] = vp0 * g0_v[t, sl] + vp1 * g1_v[t, sl]
            return carry

        lax.fori_loop(0, TPW, body, 0)
        pltpu.sync_copy(g0_v, o_hbm.at[pl.ds(w * TPW, TPW)])

    return _combine


def kernel(x, Wg, W1, b1, W2, b2):
    x2 = x.reshape(S, D)
    wg_p = jnp.pad(Wg, ((0, 0), (0, EP - E)))
    p0c, p1c, pr, eotf = _route(x2, wg_p)
    pos0 = p0c.reshape(NW, TPW)
    pos1 = p1c.reshape(NW, TPW)
    xs = _dispatch_kernel()(x2, pos0, pos1)
    ys = _ffn(eotf, xs, W1, b1.reshape(E, 1, H), W2, b2.reshape(E, 1, D))
    out = _combine_kernel()(ys, pos0, pos1, pr)
    return out.reshape(1, S, D)


# speedup vs baseline: 1.1357x; 1.1357x over previous
import functools

import jax
import jax.numpy as jnp
from jax import lax
from jax.experimental import pallas as pl
from jax.experimental.pallas import tpu as pltpu
from jax.experimental.pallas import tpu_sc as plsc

S, D, E, H = 2048, 768, 8, 1024
EP = 128
TILE = 512
P = 2 * S + E * TILE
NT = P // TILE
NC, NS, L = 2, 16, 16
NW = NC * NS
TPW = S // NW
NEG = -1e30

_INV_SQRT2 = 0.7071067811865476


def _gelu_exact(x):
    z = x * _INV_SQRT2
    az = jnp.abs(z)
    t = 1.0 / (1.0 + 0.3275911 * az)
    poly = ((((1.061405429 * t - 1.453152027) * t + 1.421413741) * t
             - 0.284496736) * t + 0.254829592) * t
    erf_abs = 1.0 - poly * jnp.exp(-az * az)
    erf = jnp.where(z < 0, -erf_abs, erf_abs)
    return 0.5 * x * (1.0 + erf)


def _route_body(x_ref, wg_ref, pos0_ref, pos1_ref, pr_ref, eot_ref):
    s = jnp.dot(x_ref[...], wg_ref[...], preferred_element_type=jnp.float32)
    lane = lax.broadcasted_iota(jnp.int32, (S, EP), 1)
    s = jnp.where(lane < E, s, NEG)
    m1 = jnp.max(s, axis=1, keepdims=True)
    i1 = jnp.min(jnp.where(s >= m1, lane, EP), axis=1, keepdims=True)
    oh1 = lane == i1
    s2 = jnp.where(oh1, NEG, s)
    m2 = jnp.max(s2, axis=1, keepdims=True)
    i2 = jnp.min(jnp.where(s2 >= m2, lane, EP), axis=1, keepdims=True)
    oh2 = lane == i2
    p1 = 1.0 / (1.0 + jnp.exp(m2 - m1))
    p2 = 1.0 - p1

    cnt = oh1.astype(jnp.float32) + oh2.astype(jnp.float32)
    G = 16
    GS = S // G
    cnt3 = cnt.reshape(G, GS, EP)
    ri = lax.broadcasted_iota(jnp.int32, (GS, GS), 0)
    ci = lax.broadcasted_iota(jnp.int32, (GS, GS), 1)
    l_incl = (ri >= ci).astype(jnp.float32)
    incl = [jnp.dot(l_incl, cnt3[g], preferred_element_type=jnp.float32)
            for g in range(G)]
    tot = jnp.concatenate([incl[g][GS - 1:GS, :] for g in range(G)], axis=0)
    ri2 = lax.broadcasted_iota(jnp.int32, (G, G), 0)
    ci2 = lax.broadcasted_iota(jnp.int32, (G, G), 1)
    m_strict = (ci2 < ri2).astype(jnp.float32)
    go = jnp.dot(m_strict, tot, preferred_element_type=jnp.float32)
    excl = jnp.concatenate(
        [go[g:g + 1, :] + incl[g] - cnt3[g] for g in range(G)], axis=0)

    c = go[G - 1:G, :] + tot[G - 1:G, :]
    pc = jnp.ceil(c * (1.0 / TILE)) * TILE
    ri3 = lax.broadcasted_iota(jnp.int32, (EP, EP), 0)
    ci3 = lax.broadcasted_iota(jnp.int32, (EP, EP), 1)
    m_lt = (ri3 < ci3).astype(jnp.float32)
    po = jnp.dot(pc, m_lt, preferred_element_type=jnp.float32)

    pos0 = jnp.sum(jnp.where(oh1, po + excl, 0.0), axis=1, keepdims=True)
    pos1 = jnp.sum(jnp.where(oh2, po + excl, 0.0), axis=1, keepdims=True)
    pos0_ref[...] = pos0.astype(jnp.int32)
    pos1_ref[...] = pos1.astype(jnp.int32)
    pr_ref[...] = jnp.where(lane < 64, p1, p2)

    jt = (lax.broadcasted_iota(jnp.int32, (EP, EP), 0) * TILE).astype(jnp.float32)
    ind = (po <= jt) & (ci3 < E)
    eot_col = jnp.sum(ind.astype(jnp.float32), axis=1, keepdims=True) - 1.0
    pt = jnp.sum(jnp.where(ci3[0:1] < E, pc, 0.0), axis=1, keepdims=True)
    eot_col = jnp.where(jt[:, 0:1] < pt, eot_col, eot_col + E)
    eot_ref[...] = jnp.broadcast_to(eot_col, (EP, 8)).astype(jnp.int32)


def _route(x2, wg_p):
    return pl.pallas_call(
        _route_body,
        out_shape=(
            jax.ShapeDtypeStruct((S, 1), jnp.int32),
            jax.ShapeDtypeStruct((S, 1), jnp.int32),
            jax.ShapeDtypeStruct((S, EP), jnp.float32),
            jax.ShapeDtypeStruct((EP, 8), jnp.int32),
        ),
    )(x2, wg_p)


@functools.cache
def _sc_mesh():
    return plsc.VectorSubcoreMesh(core_axis_name="c", subcore_axis_name="s")


@functools.cache
def _dispatch_kernel():
    @functools.partial(
        pl.kernel,
        out_type=jax.ShapeDtypeStruct((P, D), jnp.float32),
        mesh=_sc_mesh(),
        scratch_types=[
            pltpu.VMEM((TPW,), jnp.int32),
            pltpu.VMEM((TPW,), jnp.int32),
            pltpu.VMEM((TPW, D), jnp.float32),
            pltpu.SemaphoreType.DMA,
        ],
    )
    def _dispatch(x_hbm, i0_hbm, i1_hbm, xs_hbm, i0_v, i1_v, rows_v, sem):
        w = lax.axis_index("s") * NC + lax.axis_index("c")
        pltpu.sync_copy(x_hbm.at[pl.ds(w * TPW, TPW)], rows_v)
        pltpu.sync_copy(i0_hbm.at[w], i0_v)
        pltpu.sync_copy(i1_hbm.at[w], i1_v)
        c0 = pltpu.async_copy(rows_v, xs_hbm.at[i0_v], sem)
        c1 = pltpu.async_copy(rows_v, xs_hbm.at[i1_v], sem)
        c0.wait()
        c1.wait()

    return _dispatch


def _ffn_body(eot_ref, xs_ref, w1_ref, b1_ref, w2_ref, b2_ref, ys_ref):
    e = eot_ref[pl.program_id(0), 0]

    @pl.when(e < E)
    def _():
        h = jnp.dot(xs_ref[...], w1_ref[0], preferred_element_type=jnp.float32)
        h = _gelu_exact(h + b1_ref[0])
        ys_ref[...] = (
            jnp.dot(h, w2_ref[0], preferred_element_type=jnp.float32)
            + b2_ref[0])


def _safe(e):
    return e % E


def _ffn(eot, xs, W1, b1r, W2, b2r):
    grid_spec = pltpu.PrefetchScalarGridSpec(
        num_scalar_prefetch=1,
        grid=(NT,),
        in_specs=[
            pl.BlockSpec(
                (TILE, D),
                lambda i, eot: (jnp.where(eot[i, 0] < E, i, 0), 0)),
            pl.BlockSpec((1, D, H), lambda i, eot: (_safe(eot[i, 0]), 0, 0)),
            pl.BlockSpec((1, 1, H), lambda i, eot: (_safe(eot[i, 0]), 0, 0)),
            pl.BlockSpec((1, H, D), lambda i, eot: (_safe(eot[i, 0]), 0, 0)),
            pl.BlockSpec((1, 1, D), lambda i, eot: (_safe(eot[i, 0]), 0, 0)),
        ],
        out_specs=pl.BlockSpec((TILE, D), lambda i, eot: (i, 0)),
    )
    return pl.pallas_call(
        _ffn_body,
        grid_spec=grid_spec,
        out_shape=jax.ShapeDtypeStruct((P, D), jnp.float32),
    )(eot, xs, W1, b1r, W2, b2r)


@functools.cache
def _combine_kernel():
    @functools.partial(
        pl.kernel,
        out_type=jax.ShapeDtypeStruct((S, D), jnp.float32),
        mesh=_sc_mesh(),
        scratch_types=[
            pltpu.VMEM((TPW,), jnp.int32),
            pltpu.VMEM((TPW,), jnp.int32),
            pltpu.VMEM((TPW, D), jnp.float32),
            pltpu.VMEM((TPW, D), jnp.float32),
            pltpu.VMEM((TPW, EP), jnp.float32),
            pltpu.SemaphoreType.DMA,
        ],
    )
    def _combine(ys_hbm, i0_hbm, i1_hbm, pr_hbm, o_hbm,
                 i0_v, i1_v, g0_v, g1_v, pr_v, sem):
        w = lax.axis_index("s") * NC + lax.axis_index("c")
        pltpu.sync_copy(i0_hbm.at[w], i0_v)
        pltpu.sync_copy(i1_hbm.at[w], i1_v)
        pltpu.sync_copy(pr_hbm.at[pl.ds(w * TPW, TPW)], pr_v)
        c0 = pltpu.async_copy(ys_hbm.at[i0_v], g0_v, sem)
        c1 = pltpu.async_copy(ys_hbm.at[i1_v], g1_v, sem)
        c0.wait()
        c1.wait()

        def body(t, carry):
            vp0 = pr_v[t, pl.ds(0, L)]
            vp1 = pr_v[t, pl.ds(64, L)]
            for ch in range(D // L):
                sl = pl.ds(ch * L, L)
                g0_v[t, sl] = vp0 * g0_v[t, sl] + vp1 * g1_v[t, sl]
            return carry

        lax.fori_loop(0, TPW, body, 0)
        pltpu.sync_copy(g0_v, o_hbm.at[pl.ds(w * TPW, TPW)])

    return _combine


def kernel(x, Wg, W1, b1, W2, b2):
    x2 = x.reshape(S, D)
    wg_p = jnp.pad(Wg, ((0, 0), (0, EP - E)))
    p0c, p1c, pr, eotf = _route(x2, wg_p)
    pos0 = p0c.reshape(NW, TPW)
    pos1 = p1c.reshape(NW, TPW)
    xs = _dispatch_kernel()(x2, pos0, pos1)
    ys = _ffn(eotf, xs, W1, b1.reshape(E, 1, H), W2, b2.reshape(E, 1, D))
    out = _combine_kernel()(ys, pos0, pos1, pr)
    return out.reshape(1, S, D)


# speedup vs baseline: 1.1393x; 1.0032x over previous
import functools

import jax
import jax.numpy as jnp
from jax import lax
from jax.experimental import pallas as pl
from jax.experimental.pallas import tpu as pltpu
from jax.experimental.pallas import tpu_sc as plsc

S, D, E, H = 2048, 768, 8, 1024
EP = 128
TILE = 512
P = 2 * S + E * TILE
NT = P // TILE
NC, NS, L = 2, 16, 16
NW = NC * NS
TPW = S // NW
SB = TPW // 2
NEG = -1e30

_INV_SQRT2 = 0.7071067811865476


def _gelu_exact(x):
    z = x * _INV_SQRT2
    az = jnp.abs(z)
    t = 1.0 / (1.0 + 0.3275911 * az)
    poly = ((((1.061405429 * t - 1.453152027) * t + 1.421413741) * t
             - 0.284496736) * t + 0.254829592) * t
    erf_abs = 1.0 - poly * jnp.exp(-az * az)
    erf = jnp.where(z < 0, -erf_abs, erf_abs)
    return 0.5 * x * (1.0 + erf)


def _route_body(x_ref, wg_ref, pos0_ref, pos1_ref, pr_ref, eot_ref):
    s = jnp.dot(x_ref[...], wg_ref[...], preferred_element_type=jnp.float32)
    lane = lax.broadcasted_iota(jnp.int32, (S, EP), 1)
    s = jnp.where(lane < E, s, NEG)
    m1 = jnp.max(s, axis=1, keepdims=True)
    i1 = jnp.min(jnp.where(s >= m1, lane, EP), axis=1, keepdims=True)
    oh1 = lane == i1
    s2 = jnp.where(oh1, NEG, s)
    m2 = jnp.max(s2, axis=1, keepdims=True)
    i2 = jnp.min(jnp.where(s2 >= m2, lane, EP), axis=1, keepdims=True)
    oh2 = lane == i2
    p1 = 1.0 / (1.0 + jnp.exp(m2 - m1))
    p2 = 1.0 - p1

    cnt = oh1.astype(jnp.float32) + oh2.astype(jnp.float32)
    G = 16
    GS = S // G
    cnt3 = cnt.reshape(G, GS, EP)
    ri = lax.broadcasted_iota(jnp.int32, (GS, GS), 0)
    ci = lax.broadcasted_iota(jnp.int32, (GS, GS), 1)
    l_incl = (ri >= ci).astype(jnp.float32)
    incl = [jnp.dot(l_incl, cnt3[g], preferred_element_type=jnp.float32)
            for g in range(G)]
    tot = jnp.concatenate([incl[g][GS - 1:GS, :] for g in range(G)], axis=0)
    ri2 = lax.broadcasted_iota(jnp.int32, (G, G), 0)
    ci2 = lax.broadcasted_iota(jnp.int32, (G, G), 1)
    m_strict = (ci2 < ri2).astype(jnp.float32)
    go = jnp.dot(m_strict, tot, preferred_element_type=jnp.float32)
    excl = jnp.concatenate(
        [go[g:g + 1, :] + incl[g] - cnt3[g] for g in range(G)], axis=0)

    c = go[G - 1:G, :] + tot[G - 1:G, :]
    pc = jnp.ceil(c * (1.0 / TILE)) * TILE
    ri3 = lax.broadcasted_iota(jnp.int32, (EP, EP), 0)
    ci3 = lax.broadcasted_iota(jnp.int32, (EP, EP), 1)
    m_lt = (ri3 < ci3).astype(jnp.float32)
    po = jnp.dot(pc, m_lt, preferred_element_type=jnp.float32)

    pos0 = jnp.sum(jnp.where(oh1, po + excl, 0.0), axis=1, keepdims=True)
    pos1 = jnp.sum(jnp.where(oh2, po + excl, 0.0), axis=1, keepdims=True)
    pos0_ref[...] = pos0.astype(jnp.int32)
    pos1_ref[...] = pos1.astype(jnp.int32)
    pr_ref[...] = jnp.where(lane < 64, p1, p2)

    jt = (lax.broadcasted_iota(jnp.int32, (EP, EP), 0) * TILE).astype(jnp.float32)
    ind = (po <= jt) & (ci3 < E)
    eot_col = jnp.sum(ind.astype(jnp.float32), axis=1, keepdims=True) - 1.0
    pt = jnp.sum(jnp.where(ci3[0:1] < E, pc, 0.0), axis=1, keepdims=True)
    eot_col = jnp.where(jt[:, 0:1] < pt, eot_col, eot_col + E)
    eot_ref[...] = jnp.broadcast_to(eot_col, (EP, 8)).astype(jnp.int32)


def _route(x2, wg_p):
    return pl.pallas_call(
        _route_body,
        out_shape=(
            jax.ShapeDtypeStruct((S, 1), jnp.int32),
            jax.ShapeDtypeStruct((S, 1), jnp.int32),
            jax.ShapeDtypeStruct((S, EP), jnp.float32),
            jax.ShapeDtypeStruct((EP, 8), jnp.int32),
        ),
    )(x2, wg_p)


@functools.cache
def _sc_mesh():
    return plsc.VectorSubcoreMesh(core_axis_name="c", subcore_axis_name="s")


@functools.cache
def _dispatch_kernel():
    @functools.partial(
        pl.kernel,
        out_type=jax.ShapeDtypeStruct((P, D), jnp.float32),
        mesh=_sc_mesh(),
        scratch_types=[
            pltpu.VMEM((TPW,), jnp.int32),
            pltpu.VMEM((TPW,), jnp.int32),
            pltpu.VMEM((TPW, D), jnp.float32),
            pltpu.SemaphoreType.DMA,
        ],
    )
    def _dispatch(x_hbm, i0_hbm, i1_hbm, xs_hbm, i0_v, i1_v, rows_v, sem):
        w = lax.axis_index("s") * NC + lax.axis_index("c")
        pltpu.sync_copy(x_hbm.at[pl.ds(w * TPW, TPW)], rows_v)
        pltpu.sync_copy(i0_hbm.at[w], i0_v)
        pltpu.sync_copy(i1_hbm.at[w], i1_v)
        c0 = pltpu.async_copy(rows_v, xs_hbm.at[i0_v], sem)
        c1 = pltpu.async_copy(rows_v, xs_hbm.at[i1_v], sem)
        c0.wait()
        c1.wait()

    return _dispatch


def _ffn_body(eot_ref, xs_ref, w1_ref, b1_ref, w2_ref, b2_ref, ys_ref):
    e = eot_ref[pl.program_id(0), 0]

    @pl.when(e < E)
    def _():
        h = jnp.dot(xs_ref[...], w1_ref[0], preferred_element_type=jnp.float32)
        h = _gelu_exact(h + b1_ref[0])
        ys_ref[...] = (
            jnp.dot(h, w2_ref[0], preferred_element_type=jnp.float32)
            + b2_ref[0])


def _safe(e):
    return e % E


def _ffn(eot, xs, W1, b1r, W2, b2r):
    grid_spec = pltpu.PrefetchScalarGridSpec(
        num_scalar_prefetch=1,
        grid=(NT,),
        in_specs=[
            pl.BlockSpec(
                (TILE, D),
                lambda i, eot: (jnp.where(eot[i, 0] < E, i, 0), 0)),
            pl.BlockSpec((1, D, H), lambda i, eot: (_safe(eot[i, 0]), 0, 0)),
            pl.BlockSpec((1, 1, H), lambda i, eot: (_safe(eot[i, 0]), 0, 0)),
            pl.BlockSpec((1, H, D), lambda i, eot: (_safe(eot[i, 0]), 0, 0)),
            pl.BlockSpec((1, 1, D), lambda i, eot: (_safe(eot[i, 0]), 0, 0)),
        ],
        out_specs=pl.BlockSpec((TILE, D), lambda i, eot: (i, 0)),
    )
    return pl.pallas_call(
        _ffn_body,
        grid_spec=grid_spec,
        out_shape=jax.ShapeDtypeStruct((P, D), jnp.float32),
    )(eot, xs, W1, b1r, W2, b2r)


@functools.cache
def _combine_kernel():
    @functools.partial(
        pl.kernel,
        out_type=jax.ShapeDtypeStruct((S, D), jnp.float32),
        mesh=_sc_mesh(),
        scratch_types=[
            pltpu.VMEM((TPW,), jnp.int32),
            pltpu.VMEM((TPW,), jnp.int32),
            pltpu.VMEM((SB, D), jnp.float32),
            pltpu.VMEM((SB, D), jnp.float32),
            pltpu.VMEM((SB, D), jnp.float32),
            pltpu.VMEM((SB, D), jnp.float32),
            pltpu.VMEM((TPW, EP), jnp.float32),
            pltpu.SemaphoreType.DMA,
            pltpu.SemaphoreType.DMA,
            pltpu.SemaphoreType.DMA,
        ],
    )
    def _combine(ys_hbm, i0_hbm, i1_hbm, pr_hbm, o_hbm,
                 i0_v, i1_v, g0a, g1a, g0b, g1b, pr_v, sema, semb, semo):
        w = lax.axis_index("s") * NC + lax.axis_index("c")
        base = w * TPW
        pltpu.sync_copy(i0_hbm.at[w], i0_v)
        pltpu.sync_copy(i1_hbm.at[w], i1_v)
        pltpu.sync_copy(pr_hbm.at[pl.ds(base, TPW)], pr_v)
        ca0 = pltpu.async_copy(ys_hbm.at[i0_v.at[pl.ds(0, SB)]], g0a, sema)
        ca1 = pltpu.async_copy(ys_hbm.at[i1_v.at[pl.ds(0, SB)]], g1a, sema)
        cb0 = pltpu.async_copy(ys_hbm.at[i0_v.at[pl.ds(SB, SB)]], g0b, semb)
        cb1 = pltpu.async_copy(ys_hbm.at[i1_v.at[pl.ds(SB, SB)]], g1b, semb)

        def make_body(g0_v, g1_v, off):
            def body(t, carry):
                vp0 = pr_v[off + t, pl.ds(0, L)]
                vp1 = pr_v[off + t, pl.ds(64, L)]
                for ch in range(D // L):
                    sl = pl.ds(ch * L, L)
                    g0_v[t, sl] = vp0 * g0_v[t, sl] + vp1 * g1_v[t, sl]
                return carry
            return body

        ca0.wait()
        ca1.wait()
        lax.fori_loop(0, SB, make_body(g0a, g1a, 0), 0)
        oa = pltpu.async_copy(g0a, o_hbm.at[pl.ds(base, SB)], semo)
        cb0.wait()
        cb1.wait()
        lax.fori_loop(0, SB, make_body(g0b, g1b, SB), 0)
        ob = pltpu.async_copy(g0b, o_hbm.at[pl.ds(base + SB, SB)], semo)
        oa.wait()
        ob.wait()

    return _combine


def kernel(x, Wg, W1, b1, W2, b2):
    x2 = x.reshape(S, D)
    wg_p = jnp.pad(Wg, ((0, 0), (0, EP - E)))
    p0c, p1c, pr, eotf = _route(x2, wg_p)
    pos0 = p0c.reshape(NW, TPW)
    pos1 = p1c.reshape(NW, TPW)
    xs = _dispatch_kernel()(x2, pos0, pos1)
    ys = _ffn(eotf, xs, W1, b1.reshape(E, 1, H), W2, b2.reshape(E, 1, D))
    out = _combine_kernel()(ys, pos0, pos1, pr)
    return out.reshape(1, S, D)


# speedup vs baseline: 1.1562x; 1.0148x over previous
import functools

import jax
import jax.numpy as jnp
from jax import lax
from jax.experimental import pallas as pl
from jax.experimental.pallas import tpu as pltpu
from jax.experimental.pallas import tpu_sc as plsc

S, D, E, H = 2048, 768, 8, 1024
EP = 128
TILE = 512
P = 2 * S + E * TILE
NT = P // TILE
NC, NS, L = 2, 16, 16
NW = NC * NS
TPW = S // NW
SB = TPW // 2
NEG = -1e30

_INV_SQRT2 = 0.7071067811865476


def _gelu_exact(x):
    z = x * _INV_SQRT2
    az = jnp.abs(z)
    t = 1.0 / (1.0 + 0.3275911 * az)
    poly = ((((1.061405429 * t - 1.453152027) * t + 1.421413741) * t
             - 0.284496736) * t + 0.254829592) * t
    erf_abs = 1.0 - poly * jnp.exp(-az * az)
    erf = jnp.where(z < 0, -erf_abs, erf_abs)
    return 0.5 * x * (1.0 + erf)


def _route_body(x_ref, wg_ref, pos0_ref, pos1_ref, pr_ref, eot_ref):
    s = jnp.dot(x_ref[...], wg_ref[...], preferred_element_type=jnp.float32)
    lane = lax.broadcasted_iota(jnp.int32, (S, EP), 1)
    s = jnp.where(lane < E, s, NEG)
    m1 = jnp.max(s, axis=1, keepdims=True)
    i1 = jnp.min(jnp.where(s >= m1, lane, EP), axis=1, keepdims=True)
    oh1 = lane == i1
    s2 = jnp.where(oh1, NEG, s)
    m2 = jnp.max(s2, axis=1, keepdims=True)
    i2 = jnp.min(jnp.where(s2 >= m2, lane, EP), axis=1, keepdims=True)
    oh2 = lane == i2
    p1 = 1.0 / (1.0 + jnp.exp(m2 - m1))
    p2 = 1.0 - p1

    cnt = oh1.astype(jnp.float32) + oh2.astype(jnp.float32)
    G = 16
    GS = S // G
    cnt3 = cnt.reshape(G, GS, EP)
    ri = lax.broadcasted_iota(jnp.int32, (GS, GS), 0)
    ci = lax.broadcasted_iota(jnp.int32, (GS, GS), 1)
    l_incl = (ri >= ci).astype(jnp.float32)
    incl = [jnp.dot(l_incl, cnt3[g], preferred_element_type=jnp.float32)
            for g in range(G)]
    tot = jnp.concatenate([incl[g][GS - 1:GS, :] for g in range(G)], axis=0)
    ri2 = lax.broadcasted_iota(jnp.int32, (G, G), 0)
    ci2 = lax.broadcasted_iota(jnp.int32, (G, G), 1)
    m_strict = (ci2 < ri2).astype(jnp.float32)
    go = jnp.dot(m_strict, tot, preferred_element_type=jnp.float32)
    excl = jnp.concatenate(
        [go[g:g + 1, :] + incl[g] - cnt3[g] for g in range(G)], axis=0)

    c = go[G - 1:G, :] + tot[G - 1:G, :]
    pc = jnp.ceil(c * (1.0 / TILE)) * TILE
    ri3 = lax.broadcasted_iota(jnp.int32, (EP, EP), 0)
    ci3 = lax.broadcasted_iota(jnp.int32, (EP, EP), 1)
    m_lt = (ri3 < ci3).astype(jnp.float32)
    po = jnp.dot(pc, m_lt, preferred_element_type=jnp.float32)

    pos0 = jnp.sum(jnp.where(oh1, po + excl, 0.0), axis=1, keepdims=True)
    pos1 = jnp.sum(jnp.where(oh2, po + excl, 0.0), axis=1, keepdims=True)
    pos0_ref[...] = pos0.astype(jnp.int32)
    pos1_ref[...] = pos1.astype(jnp.int32)
    pr_ref[...] = jnp.where(lane < 64, p1, p2)

    jt = (lax.broadcasted_iota(jnp.int32, (EP, EP), 0) * TILE).astype(jnp.float32)
    ind = (po <= jt) & (ci3 < E)
    eot_col = jnp.sum(ind.astype(jnp.float32), axis=1, keepdims=True) - 1.0
    pt = jnp.sum(jnp.where(ci3[0:1] < E, pc, 0.0), axis=1, keepdims=True)
    eot_col = jnp.where(jt[:, 0:1] < pt, eot_col, eot_col + E)
    eot_ref[...] = jnp.broadcast_to(eot_col, (EP, 8)).astype(jnp.int32)


def _route(x2, wg_p):
    return pl.pallas_call(
        _route_body,
        out_shape=(
            jax.ShapeDtypeStruct((S, 1), jnp.int32),
            jax.ShapeDtypeStruct((S, 1), jnp.int32),
            jax.ShapeDtypeStruct((S, EP), jnp.float32),
            jax.ShapeDtypeStruct((EP, 8), jnp.int32),
        ),
    )(x2, wg_p)


@functools.cache
def _sc_mesh():
    return plsc.VectorSubcoreMesh(core_axis_name="c", subcore_axis_name="s")


@functools.cache
def _dispatch_kernel():
    @functools.partial(
        pl.kernel,
        out_type=jax.ShapeDtypeStruct((P, D), jnp.float32),
        mesh=_sc_mesh(),
        scratch_types=[
            pltpu.VMEM((2, SB), jnp.int32),
            pltpu.VMEM((2, SB), jnp.int32),
            pltpu.VMEM((SB, D), jnp.float32),
            pltpu.VMEM((SB, D), jnp.float32),
            pltpu.SemaphoreType.DMA,
            pltpu.SemaphoreType.DMA,
            pltpu.SemaphoreType.DMA,
        ],
    )
    def _dispatch(x_hbm, i0_hbm, i1_hbm, xs_hbm,
                  i0_v, i1_v, ra, rb, semld, sema, semb):
        w = lax.axis_index("s") * NC + lax.axis_index("c")
        base = w * TPW
        la = pltpu.async_copy(x_hbm.at[pl.ds(base, SB)], ra, semld)
        lb = pltpu.async_copy(x_hbm.at[pl.ds(base + SB, SB)], rb, semld)
        pltpu.sync_copy(i0_hbm.at[w], i0_v)
        pltpu.sync_copy(i1_hbm.at[w], i1_v)
        la.wait()
        a0 = pltpu.async_copy(ra, xs_hbm.at[i0_v.at[0]], sema)
        a1 = pltpu.async_copy(ra, xs_hbm.at[i1_v.at[0]], sema)
        lb.wait()
        b0 = pltpu.async_copy(rb, xs_hbm.at[i0_v.at[1]], semb)
        b1 = pltpu.async_copy(rb, xs_hbm.at[i1_v.at[1]], semb)
        a0.wait()
        a1.wait()
        b0.wait()
        b1.wait()

    return _dispatch


def _ffn_body(eot_ref, xs_ref, w1_ref, b1_ref, w2_ref, b2_ref, ys_ref):
    e = eot_ref[pl.program_id(0), 0]

    @pl.when(e < E)
    def _():
        h = jnp.dot(xs_ref[...], w1_ref[0], preferred_element_type=jnp.float32)
        h = _gelu_exact(h + b1_ref[0])
        ys_ref[...] = (
            jnp.dot(h, w2_ref[0], preferred_element_type=jnp.float32)
            + b2_ref[0])


def _safe(e):
    return e % E


def _ffn(eot, xs, W1, b1r, W2, b2r):
    grid_spec = pltpu.PrefetchScalarGridSpec(
        num_scalar_prefetch=1,
        grid=(NT,),
        in_specs=[
            pl.BlockSpec(
                (TILE, D),
                lambda i, eot: (jnp.where(eot[i, 0] < E, i, 0), 0)),
            pl.BlockSpec((1, D, H), lambda i, eot: (_safe(eot[i, 0]), 0, 0)),
            pl.BlockSpec((1, 1, H), lambda i, eot: (_safe(eot[i, 0]), 0, 0)),
            pl.BlockSpec((1, H, D), lambda i, eot: (_safe(eot[i, 0]), 0, 0)),
            pl.BlockSpec((1, 1, D), lambda i, eot: (_safe(eot[i, 0]), 0, 0)),
        ],
        out_specs=pl.BlockSpec((TILE, D), lambda i, eot: (i, 0)),
    )
    return pl.pallas_call(
        _ffn_body,
        grid_spec=grid_spec,
        out_shape=jax.ShapeDtypeStruct((P, D), jnp.float32),
    )(eot, xs, W1, b1r, W2, b2r)


@functools.cache
def _combine_kernel():
    @functools.partial(
        pl.kernel,
        out_type=jax.ShapeDtypeStruct((S, D), jnp.float32),
        mesh=_sc_mesh(),
        scratch_types=[
            pltpu.VMEM((2, SB), jnp.int32),
            pltpu.VMEM((2, SB), jnp.int32),
            pltpu.VMEM((SB, D), jnp.float32),
            pltpu.VMEM((SB, D), jnp.float32),
            pltpu.VMEM((SB, D), jnp.float32),
            pltpu.VMEM((SB, D), jnp.float32),
            pltpu.VMEM((TPW, EP), jnp.float32),
            pltpu.SemaphoreType.DMA,
            pltpu.SemaphoreType.DMA,
            pltpu.SemaphoreType.DMA,
        ],
    )
    def _combine(ys_hbm, i0_hbm, i1_hbm, pr_hbm, o_hbm,
                 i0_v, i1_v, g0a, g1a, g0b, g1b, pr_v, sema, semb, semo):
        w = lax.axis_index("s") * NC + lax.axis_index("c")
        base = w * TPW
        pltpu.sync_copy(i0_hbm.at[w], i0_v)
        pltpu.sync_copy(i1_hbm.at[w], i1_v)
        pltpu.sync_copy(pr_hbm.at[pl.ds(base, TPW)], pr_v)
        ca0 = pltpu.async_copy(ys_hbm.at[i0_v.at[0]], g0a, sema)
        ca1 = pltpu.async_copy(ys_hbm.at[i1_v.at[0]], g1a, sema)
        cb0 = pltpu.async_copy(ys_hbm.at[i0_v.at[1]], g0b, semb)
        cb1 = pltpu.async_copy(ys_hbm.at[i1_v.at[1]], g1b, semb)

        def make_body(g0_v, g1_v, off):
            def body(t, carry):
                vp0 = pr_v[off + t, pl.ds(0, L)]
                vp1 = pr_v[off + t, pl.ds(64, L)]
                for ch in range(D // L):
                    sl = pl.ds(ch * L, L)
                    g0_v[t, sl] = vp0 * g0_v[t, sl] + vp1 * g1_v[t, sl]
                return carry
            return body

        ca0.wait()
        ca1.wait()
        lax.fori_loop(0, SB, make_body(g0a, g1a, 0), 0)
        oa = pltpu.async_copy(g0a, o_hbm.at[pl.ds(base, SB)], semo)
        cb0.wait()
        cb1.wait()
        lax.fori_loop(0, SB, make_body(g0b, g1b, SB), 0)
        ob = pltpu.async_copy(g0b, o_hbm.at[pl.ds(base + SB, SB)], semo)
        oa.wait()
        ob.wait()

    return _combine


def kernel(x, Wg, W1, b1, W2, b2):
    x2 = x.reshape(S, D)
    wg_p = jnp.pad(Wg, ((0, 0), (0, EP - E)))
    p0c, p1c, pr, eotf = _route(x2, wg_p)
    pos0 = p0c.reshape(NW, 2, SB)
    pos1 = p1c.reshape(NW, 2, SB)
    xs = _dispatch_kernel()(x2, pos0, pos1)
    ys = _ffn(eotf, xs, W1, b1.reshape(E, 1, H), W2, b2.reshape(E, 1, D))
    out = _combine_kernel()(ys, pos0, pos1, pr)
    return out.reshape(1, S, D)
